# baseline (speedup 1.0000x reference)
import functools

import jax
import jax.numpy as jnp
from jax import lax
from jax.experimental import pallas as pl
from jax.experimental.pallas import tpu as pltpu
from jax.experimental.pallas import tpu_sc as plsc

B, L = 4096, 200
N = B * L
NC, NS, LANES = 2, 16, 16
NW = NC * NS
NTOK = N // NW
C = 400
NCHUNK = NTOK // C
NG = C // LANES
NPAIR = NCHUNK // 2
NF = 11 * 8 * 3 * 3
NFPAD = 1024


def _ge_count(x, step, n):
    acc = jnp.zeros_like(x)
    for k in range(1, n + 1):
        acc = acc + (x >= step * k).astype(jnp.int32)
    return acc


def _body(ii, pp, ss, cc, tt, el, lg,
          item_t, part_t, sec_t, corr_t, time_t, w,
          out,
          iidx_v, pp_v, ss_v, cc_v, tt_v, el_v, lg_v, fidx_v,
          part_v, sec_v, corr_v, time_v, w_v,
          rows_v, sm_v, eg_v, f_sh,
          in_sem, gat_sem, sf_sem, out_sem, out2_sem):
    wid = lax.axis_index("s") * NC + lax.axis_index("c")
    base0 = wid * NTOK

    pltpu.sync_copy(part_t, part_v)
    pltpu.sync_copy(sec_t, sec_v)
    pltpu.sync_copy(corr_t, corr_v)
    pltpu.sync_copy(time_t, time_v)
    pltpu.sync_copy(w, w_v)
    wvec = w_v[...]

    sid0 = lax.axis_index("s")

    def fgroup(j, carry):
        g = sid0 + j * NS
        o = g * LANES
        f = jnp.minimum(lax.iota(jnp.int32, LANES) + o, NF - 1)
        p = _ge_count(f, 72, 10)
        r = f - p * 72
        s = _ge_count(r, 9, 7)
        r2 = r - s * 9
        c = _ge_count(r2, 3, 2)
        t = r2 - c * 3
        offs = lax.iota(jnp.int32, LANES)

        def flush(pairs):
            for v, col in pairs:
                plsc.store_scatter(
                    sm_v[0], [offs, jnp.full((LANES,), col, jnp.int32)], v)

        for d0 in range(0, 16, 8):
            pairs = []
            for d in range(d0, d0 + 8):
                dcol = jnp.full((LANES,), d, jnp.int32)
                pairs.append((plsc.load_gather(part_v, [p, dcol]), d))
                pairs.append((plsc.load_gather(sec_v, [s, dcol]), d + 16))
            flush(pairs)
        pairs = []
        for d in range(8):
            dcol = jnp.full((LANES,), d, jnp.int32)
            pairs.append((plsc.load_gather(corr_v, [c, dcol]), d + 32))
            pairs.append((plsc.load_gather(time_v, [t, dcol]), d + 40))
        flush(pairs)
        pltpu.sync_copy(sm_v[0].at[pl.ds(0, LANES)], f_sh.at[pl.ds(o, LANES)])
        return carry

    lax.fori_loop(0, NFPAD // (LANES * NS), fgroup, 0)

    plsc.subcore_barrier()

    def in_copies(k, b):
        base = base0 + k * C
        s = in_sem[b]
        return [
            pltpu.make_async_copy(ii.at[pl.ds(base, C)], iidx_v[b], s),
            pltpu.make_async_copy(pp.at[pl.ds(base, C)], pp_v[b], s),
            pltpu.make_async_copy(ss.at[pl.ds(base, C)], ss_v[b], s),
            pltpu.make_async_copy(cc.at[pl.ds(base, C)], cc_v[b], s),
            pltpu.make_async_copy(tt.at[pl.ds(base, C)], tt_v[b], s),
            pltpu.make_async_copy(el.at[pl.ds(base, C)], el_v[b], s),
            pltpu.make_async_copy(lg.at[pl.ds(base, C)], lg_v[b], s),
        ]

    def issue_in(k, b):
        for c in in_copies(k, b):
            c.start()

    def wait_in(k, b):
        for c in in_copies(k, b):
            c.wait()

    def gat_copy(b):
        return pltpu.make_async_copy(
            item_t.at[iidx_v[b]], rows_v[b], gat_sem[b])

    def sf_copy(b):
        return pltpu.make_async_copy(
            f_sh.at[fidx_v[b]], sm_v[b], sf_sem[b])

    def rows_copy(k, b):
        base = base0 + k * C
        return pltpu.make_async_copy(
            rows_v[b], out.at[pl.ds(base, C), pl.ds(0, 64)], out_sem[b])

    def sm_copy(k, b):
        base = base0 + k * C
        return pltpu.make_async_copy(
            sm_v[b], out.at[pl.ds(base, C), pl.ds(64, 48)], out2_sem[b])

    def eg_copy(k, b):
        base = base0 + k * C
        return pltpu.make_async_copy(
            eg_v[b].at[:, pl.ds(0, 16)],
            out.at[pl.ds(base, C), pl.ds(112, 16)], out2_sem[b])

    def comp_fidx(b):
        def group(g, gcarry):
            o = g * LANES
            pid = pp_v[b][pl.ds(o, LANES)]
            sid = ss_v[b][pl.ds(o, LANES)]
            cid = cc_v[b][pl.ds(o, LANES)]
            tid = tt_v[b][pl.ds(o, LANES)]
            fidx_v[b][pl.ds(o, LANES)] = pid * 72 + sid * 9 + cid * 3 + tid
            return gcarry

        lax.fori_loop(0, NG, group, 0)

    def comp_ellag(b):
        def group(g, gcarry):
            o = g * LANES
            offs = lax.iota(jnp.int32, LANES) + o
            elv = el_v[b][pl.ds(o, LANES)]
            lgv = lg_v[b][pl.ds(o, LANES)]
            pairs = []
            for d in range(8):
                pairs.append((elv * wvec[d], d))
                pairs.append((lgv * wvec[8 + d], d + 8))
            for v, col in pairs:
                plsc.store_scatter(
                    eg_v[b], [offs, jnp.full((LANES,), col, jnp.int32)], v)
            return gcarry

        lax.fori_loop(0, NG, group, 0)

    issue_in(0, 0)
    wait_in(0, 0)
    gat_copy(0).start()
    comp_fidx(0)
    sf_copy(0).start()

    def pair(i, carry):
        kk = 2 * i
        for b in range(2):
            k = kk + b
            q = 1 - b

            @pl.when(k + 1 < NCHUNK)
            def _():
                issue_in(k + 1, q)

            comp_ellag(b)

            @pl.when(k + 1 < NCHUNK)
            def _():
                wait_in(k + 1, q)

                @pl.when(k >= 1)
                def _():
                    rows_copy(k - 1, q).wait()

                gat_copy(q).start()
                comp_fidx(q)

                @pl.when(k >= 1)
                def _():
                    sm_copy(k - 1, q).wait()
                    eg_copy(k - 1, q).wait()

                sf_copy(q).start()

            gat_copy(b).wait()
            rows_copy(k, b).start()
            sf_copy(b).wait()
            sm_copy(k, b).start()
            eg_copy(k, b).start()
        return carry

    lax.fori_loop(0, NPAIR, pair, 0)
    for k, b in ((NCHUNK - 2, 0), (NCHUNK - 1, 1)):
        rows_copy(k, b).wait()
        sm_copy(k, b).wait()
        eg_copy(k, b).wait()


@jax.jit
def _run(ii, pp, ss, cc, tt, el, lg, item_t, part_t, sec_t, corr_t, time_t, w):
    mesh = plsc.VectorSubcoreMesh(core_axis_name="c", subcore_axis_name="s")
    dbl = lambda *a: [pltpu.VMEM(*a), pltpu.VMEM(*a)]
    f = pl.kernel(
        _body,
        out_type=jax.ShapeDtypeStruct((N, 128), jnp.float32),
        mesh=mesh,
        compiler_params=pltpu.CompilerParams(use_tc_tiling_on_sc=False,
                                            needs_layout_passes=False),
        scratch_types=[
            dbl((C,), jnp.int32),
            dbl((C,), jnp.int32),
            dbl((C,), jnp.int32),
            dbl((C,), jnp.int32),
            dbl((C,), jnp.int32),
            dbl((C,), jnp.float32),
            dbl((C,), jnp.float32),
            dbl((C,), jnp.int32),
            pltpu.VMEM((11, 17), jnp.float32),
            pltpu.VMEM((8, 17), jnp.float32),
            pltpu.VMEM((3, 9), jnp.float32),
            pltpu.VMEM((3, 9), jnp.float32),
            pltpu.VMEM((16,), jnp.float32),
            dbl((C, 64), jnp.float32),
            dbl((C, 48), jnp.float32),
            dbl((C, 17), jnp.float32),
            pltpu.VMEM_SHARED((NFPAD, 48), jnp.float32),
            [pltpu.SemaphoreType.DMA, pltpu.SemaphoreType.DMA],
            [pltpu.SemaphoreType.DMA, pltpu.SemaphoreType.DMA],
            [pltpu.SemaphoreType.DMA, pltpu.SemaphoreType.DMA],
            [pltpu.SemaphoreType.DMA, pltpu.SemaphoreType.DMA],
            [pltpu.SemaphoreType.DMA, pltpu.SemaphoreType.DMA],
        ],
    )
    return f(ii, pp, ss, cc, tt, el, lg, item_t, part_t, sec_t, corr_t, time_t, w)


def kernel(item_id, part_id, section, is_correct, timeliness,
           elapsed_time_norm, lag_time_norm,
           item_table, part_table, section_table,
           is_correct_table, timeliness_table, W_elapsed, W_lag):
    ii = item_id.reshape(N).astype(jnp.int32)
    pp = part_id.reshape(N).astype(jnp.int32)
    ss = section.reshape(N).astype(jnp.int32)
    cc = is_correct.reshape(N).astype(jnp.int32)
    tt = timeliness.reshape(N).astype(jnp.int32)
    el = elapsed_time_norm.reshape(N)
    lg = lag_time_norm.reshape(N)
    w = jnp.concatenate([W_elapsed.reshape(8), W_lag.reshape(8)])
    part_p = jnp.pad(part_table, ((0, 0), (0, 1)))
    sec_p = jnp.pad(section_table, ((0, 0), (0, 1)))
    corr_p = jnp.pad(is_correct_table, ((0, 0), (0, 1)))
    time_p = jnp.pad(timeliness_table, ((0, 0), (0, 1)))
    out = _run(ii, pp, ss, cc, tt, el, lg,
               item_table, part_p, sec_p, corr_p, time_p, w)
    return out.reshape(B, L, 128)


# speedup vs baseline: 1.0113x; 1.0113x over previous
import functools

import jax
import jax.numpy as jnp
from jax import lax
from jax.experimental import pallas as pl
from jax.experimental.pallas import tpu as pltpu
from jax.experimental.pallas import tpu_sc as plsc

B, L = 4096, 200
N = B * L
NC, NS, LANES = 2, 16, 16
NW = NC * NS
NTOK = N // NW
C = 400
NCHUNK = NTOK // C
NG = C // LANES
NPAIR = NCHUNK // 2
NF = 11 * 8 * 3 * 3
NFPAD = 1024


def _ge_count(x, step, n):
    acc = jnp.zeros_like(x)
    for k in range(1, n + 1):
        acc = acc + (x >= step * k).astype(jnp.int32)
    return acc


def _body(ii, pp, ss, cc, tt, el, lg,
          item_t, part_t, sec_t, corr_t, time_t, w,
          out,
          iidx_v, pp_v, ss_v, cc_v, tt_v, el_v, lg_v, fidx_v,
          part_v, sec_v, corr_v, time_v, w_v,
          rows_v, sm_v, eg_v, f_sh,
          in_sem, gat_sem, sf_sem, out_sem, out2_sem):
    wid = lax.axis_index("s") * NC + lax.axis_index("c")
    base0 = wid * NTOK

    pltpu.sync_copy(part_t, part_v)
    pltpu.sync_copy(sec_t, sec_v)
    pltpu.sync_copy(corr_t, corr_v)
    pltpu.sync_copy(time_t, time_v)
    pltpu.sync_copy(w, w_v)
    wvec = w_v[...]

    sid0 = lax.axis_index("s")

    def fgroup(j, carry):
        g = sid0 + j * NS
        o = g * LANES
        f = jnp.minimum(lax.iota(jnp.int32, LANES) + o, NF - 1)
        p = _ge_count(f, 72, 10)
        r = f - p * 72
        s = _ge_count(r, 9, 7)
        r2 = r - s * 9
        c = _ge_count(r2, 3, 2)
        t = r2 - c * 3
        offs = lax.iota(jnp.int32, LANES)

        def flush(pairs):
            for v, col in pairs:
                plsc.store_scatter(
                    sm_v[0], [offs, jnp.full((LANES,), col, jnp.int32)], v)

        for d0 in range(0, 16, 8):
            pairs = []
            for d in range(d0, d0 + 8):
                dcol = jnp.full((LANES,), d, jnp.int32)
                pairs.append((plsc.load_gather(part_v, [p, dcol]), d))
                pairs.append((plsc.load_gather(sec_v, [s, dcol]), d + 16))
            flush(pairs)
        pairs = []
        for d in range(8):
            dcol = jnp.full((LANES,), d, jnp.int32)
            pairs.append((plsc.load_gather(corr_v, [c, dcol]), d + 32))
            pairs.append((plsc.load_gather(time_v, [t, dcol]), d + 40))
        flush(pairs)
        pltpu.sync_copy(sm_v[0].at[pl.ds(0, LANES)], f_sh.at[pl.ds(o, LANES)])
        return carry

    lax.fori_loop(0, NFPAD // (LANES * NS), fgroup, 0)

    plsc.subcore_barrier()

    def in_copies(k, b):
        base = base0 + k * C
        s = in_sem[b]
        return [
            pltpu.make_async_copy(ii.at[pl.ds(base, C)], iidx_v[b], s),
            pltpu.make_async_copy(pp.at[pl.ds(base, C)], pp_v[b], s),
            pltpu.make_async_copy(ss.at[pl.ds(base, C)], ss_v[b], s),
            pltpu.make_async_copy(cc.at[pl.ds(base, C)], cc_v[b], s),
            pltpu.make_async_copy(tt.at[pl.ds(base, C)], tt_v[b], s),
            pltpu.make_async_copy(el.at[pl.ds(base, C)], el_v[b], s),
            pltpu.make_async_copy(lg.at[pl.ds(base, C)], lg_v[b], s),
        ]

    def issue_in(k, b):
        for c in in_copies(k, b):
            c.start()

    def wait_in(k, b):
        for c in in_copies(k, b):
            c.wait()

    def gat_copy(b):
        return pltpu.make_async_copy(
            item_t.at[iidx_v[b]], rows_v[b], gat_sem[b])

    def sf_copy(b):
        return pltpu.make_async_copy(
            f_sh.at[fidx_v[b]], sm_v[b], sf_sem[b])

    def rows_copy(k, b):
        base = base0 + k * C
        return pltpu.make_async_copy(
            rows_v[b], out.at[pl.ds(base, C), pl.ds(0, 64)], out_sem[b])

    def sm_copy(k, b):
        base = base0 + k * C
        return pltpu.make_async_copy(
            sm_v[b], out.at[pl.ds(base, C), pl.ds(64, 48)], out2_sem[b])

    def eg_copy(k, b):
        base = base0 + k * C
        return pltpu.make_async_copy(
            eg_v[b].at[:, pl.ds(0, 16)],
            out.at[pl.ds(base, C), pl.ds(112, 16)], out2_sem[b])

    def comp_fidx(b):
        def group(g, gcarry):
            o = g * LANES
            pid = pp_v[b][pl.ds(o, LANES)]
            sid = ss_v[b][pl.ds(o, LANES)]
            cid = cc_v[b][pl.ds(o, LANES)]
            tid = tt_v[b][pl.ds(o, LANES)]
            fidx_v[b][pl.ds(o, LANES)] = pid * 72 + sid * 9 + cid * 3 + tid
            return gcarry

        lax.fori_loop(0, NG, group, 0)

    def comp_ellag(b):
        def group(g, gcarry):
            o = g * LANES
            offs = lax.iota(jnp.int32, LANES) + o
            elv = el_v[b][pl.ds(o, LANES)]
            lgv = lg_v[b][pl.ds(o, LANES)]
            pairs = []
            for d in range(8):
                pairs.append((elv * wvec[d], d))
                pairs.append((lgv * wvec[8 + d], d + 8))
            for v, col in pairs:
                plsc.store_scatter(
                    eg_v[b], [offs, jnp.full((LANES,), col, jnp.int32)], v)
            return gcarry

        lax.fori_loop(0, NG, group, 0)

    issue_in(0, 0)
    wait_in(0, 0)
    gat_copy(0).start()

    def pair(i, carry):
        kk = 2 * i
        for b in range(2):
            k = kk + b
            q = 1 - b

            @pl.when(k >= 2)
            def _():
                sm_copy(k - 2, b).wait()
                eg_copy(k - 2, b).wait()

            @pl.when(k + 1 < NCHUNK)
            def _():
                issue_in(k + 1, q)

            comp_fidx(b)
            sf_copy(b).start()
            comp_ellag(b)

            @pl.when(k + 1 < NCHUNK)
            def _():
                wait_in(k + 1, q)

                @pl.when(k >= 1)
                def _():
                    rows_copy(k - 1, q).wait()

                gat_copy(q).start()

            gat_copy(b).wait()
            rows_copy(k, b).start()
            sf_copy(b).wait()
            sm_copy(k, b).start()
            eg_copy(k, b).start()
        return carry

    lax.fori_loop(0, NPAIR, pair, 0)
    for k, b in ((NCHUNK - 2, 0), (NCHUNK - 1, 1)):
        rows_copy(k, b).wait()
        sm_copy(k, b).wait()
        eg_copy(k, b).wait()


@jax.jit
def _run(ii, pp, ss, cc, tt, el, lg, item_t, part_t, sec_t, corr_t, time_t, w):
    mesh = plsc.VectorSubcoreMesh(core_axis_name="c", subcore_axis_name="s")
    dbl = lambda *a: [pltpu.VMEM(*a), pltpu.VMEM(*a)]
    f = pl.kernel(
        _body,
        out_type=jax.ShapeDtypeStruct((N, 128), jnp.float32),
        mesh=mesh,
        compiler_params=pltpu.CompilerParams(use_tc_tiling_on_sc=False,
                                            needs_layout_passes=False),
        scratch_types=[
            dbl((C,), jnp.int32),
            dbl((C,), jnp.int32),
            dbl((C,), jnp.int32),
            dbl((C,), jnp.int32),
            dbl((C,), jnp.int32),
            dbl((C,), jnp.float32),
            dbl((C,), jnp.float32),
            dbl((C,), jnp.int32),
            pltpu.VMEM((11, 17), jnp.float32),
            pltpu.VMEM((8, 17), jnp.float32),
            pltpu.VMEM((3, 9), jnp.float32),
            pltpu.VMEM((3, 9), jnp.float32),
            pltpu.VMEM((16,), jnp.float32),
            dbl((C, 64), jnp.float32),
            dbl((C, 48), jnp.float32),
            dbl((C, 17), jnp.float32),
            pltpu.VMEM_SHARED((NFPAD, 48), jnp.float32),
            [pltpu.SemaphoreType.DMA, pltpu.SemaphoreType.DMA],
            [pltpu.SemaphoreType.DMA, pltpu.SemaphoreType.DMA],
            [pltpu.SemaphoreType.DMA, pltpu.SemaphoreType.DMA],
            [pltpu.SemaphoreType.DMA, pltpu.SemaphoreType.DMA],
            [pltpu.SemaphoreType.DMA, pltpu.SemaphoreType.DMA],
        ],
    )
    return f(ii, pp, ss, cc, tt, el, lg, item_t, part_t, sec_t, corr_t, time_t, w)


def kernel(item_id, part_id, section, is_correct, timeliness,
           elapsed_time_norm, lag_time_norm,
           item_table, part_table, section_table,
           is_correct_table, timeliness_table, W_elapsed, W_lag):
    ii = item_id.reshape(N).astype(jnp.int32)
    pp = part_id.reshape(N).astype(jnp.int32)
    ss = section.reshape(N).astype(jnp.int32)
    cc = is_correct.reshape(N).astype(jnp.int32)
    tt = timeliness.reshape(N).astype(jnp.int32)
    el = elapsed_time_norm.reshape(N)
    lg = lag_time_norm.reshape(N)
    w = jnp.concatenate([W_elapsed.reshape(8), W_lag.reshape(8)])
    part_p = jnp.pad(part_table, ((0, 0), (0, 1)))
    sec_p = jnp.pad(section_table, ((0, 0), (0, 1)))
    corr_p = jnp.pad(is_correct_table, ((0, 0), (0, 1)))
    time_p = jnp.pad(timeliness_table, ((0, 0), (0, 1)))
    out = _run(ii, pp, ss, cc, tt, el, lg,
               item_table, part_p, sec_p, corr_p, time_p, w)
    return out.reshape(B, L, 128)


# speedup vs baseline: 1.1573x; 1.1444x over previous
import functools

import jax
import jax.numpy as jnp
from jax import lax
from jax.experimental import pallas as pl
from jax.experimental.pallas import tpu as pltpu
from jax.experimental.pallas import tpu_sc as plsc

B, L = 4096, 200
N = B * L
NC, NS, LANES = 2, 16, 16
NW = NC * NS
NTOK = N // NW
C = 400
NCHUNK = NTOK // C
NG = C // LANES
NPAIR = NCHUNK // 2
NF = 11 * 8 * 3 * 3
NFPAD = 1024


def _ge_count(x, step, n):
    acc = jnp.zeros_like(x)
    for k in range(1, n + 1):
        acc = acc + (x >= step * k).astype(jnp.int32)
    return acc


def _body(ii, pp, ss, cc, tt, el, lg,
          item_t, part_t, sec_t, corr_t, time_t, w,
          out,
          iidx_v, pp_v, ss_v, cc_v, tt_v, el_v, lg_v, fidx_v,
          part_v, sec_v, corr_v, time_v, w_v,
          rows_v, sm_v, f_sh,
          in_sem, gat_sem, sf_sem, out_sem, out2_sem):
    wid = lax.axis_index("s") * NC + lax.axis_index("c")
    base0 = wid * NTOK

    pltpu.sync_copy(part_t, part_v)
    pltpu.sync_copy(sec_t, sec_v)
    pltpu.sync_copy(corr_t, corr_v)
    pltpu.sync_copy(time_t, time_v)
    pltpu.sync_copy(w, w_v)
    wvec = w_v[...]

    sid0 = lax.axis_index("s")
    zofs = lax.iota(jnp.int32, LANES)
    for d in range(16):
        plsc.store_scatter(sm_v[0], [zofs, jnp.full((LANES,), 48 + d, jnp.int32)],
                           jnp.zeros((LANES,), jnp.float32))

    def fgroup(j, carry):
        g = sid0 + j * NS
        o = g * LANES
        f = jnp.minimum(lax.iota(jnp.int32, LANES) + o, NF - 1)
        p = _ge_count(f, 72, 10)
        r = f - p * 72
        s = _ge_count(r, 9, 7)
        r2 = r - s * 9
        c = _ge_count(r2, 3, 2)
        t = r2 - c * 3
        offs = lax.iota(jnp.int32, LANES)

        def flush(pairs):
            for v, col in pairs:
                plsc.store_scatter(
                    sm_v[0], [offs, jnp.full((LANES,), col, jnp.int32)], v)

        for d0 in range(0, 16, 8):
            pairs = []
            for d in range(d0, d0 + 8):
                dcol = jnp.full((LANES,), d, jnp.int32)
                pairs.append((plsc.load_gather(part_v, [p, dcol]), d))
                pairs.append((plsc.load_gather(sec_v, [s, dcol]), d + 16))
            flush(pairs)
        pairs = []
        for d in range(8):
            dcol = jnp.full((LANES,), d, jnp.int32)
            pairs.append((plsc.load_gather(corr_v, [c, dcol]), d + 32))
            pairs.append((plsc.load_gather(time_v, [t, dcol]), d + 40))
        flush(pairs)
        pltpu.sync_copy(sm_v[0].at[pl.ds(0, LANES)], f_sh.at[pl.ds(o, LANES)])
        return carry

    lax.fori_loop(0, NFPAD // (LANES * NS), fgroup, 0)

    plsc.subcore_barrier()

    def in_copies(k, b):
        base = base0 + k * C
        s = in_sem[b]
        return [
            pltpu.make_async_copy(ii.at[pl.ds(base, C)], iidx_v[b], s),
            pltpu.make_async_copy(pp.at[pl.ds(base, C)], pp_v[b], s),
            pltpu.make_async_copy(ss.at[pl.ds(base, C)], ss_v[b], s),
            pltpu.make_async_copy(cc.at[pl.ds(base, C)], cc_v[b], s),
            pltpu.make_async_copy(tt.at[pl.ds(base, C)], tt_v[b], s),
            pltpu.make_async_copy(el.at[pl.ds(base, C)], el_v[b], s),
            pltpu.make_async_copy(lg.at[pl.ds(base, C)], lg_v[b], s),
        ]

    def issue_in(k, b):
        for c in in_copies(k, b):
            c.start()

    def wait_in(k, b):
        for c in in_copies(k, b):
            c.wait()

    def gat_copy(b):
        return pltpu.make_async_copy(
            item_t.at[iidx_v[b]], rows_v[b], gat_sem[b])

    def sf_copy(b):
        return pltpu.make_async_copy(
            f_sh.at[fidx_v[b]], sm_v[b], sf_sem[b])

    def rows_copy(k, b):
        base = base0 + k * C
        return pltpu.make_async_copy(
            rows_v[b], out.at[pl.ds(base, C), pl.ds(0, 64)], out_sem[b])

    def sm_copy(k, b):
        base = base0 + k * C
        return pltpu.make_async_copy(
            sm_v[b], out.at[pl.ds(base, C), pl.ds(64, 64)], out2_sem[b])

    def comp_fidx(b):
        def group(g, gcarry):
            o = g * LANES
            pid = pp_v[b][pl.ds(o, LANES)]
            sid = ss_v[b][pl.ds(o, LANES)]
            cid = cc_v[b][pl.ds(o, LANES)]
            tid = tt_v[b][pl.ds(o, LANES)]
            fidx_v[b][pl.ds(o, LANES)] = pid * 72 + sid * 9 + cid * 3 + tid
            return gcarry

        lax.fori_loop(0, NG, group, 0)

    m8 = lax.iota(jnp.int32, LANES) < 8
    col48 = lax.iota(jnp.int32, LANES) + 48

    def insert_ellag(b):
        def group(g, gcarry):
            o = g * LANES
            elv = el_v[b][pl.ds(o, LANES)]
            lgv = lg_v[b][pl.ds(o, LANES)]
            for j in range(LANES):
                mix = jnp.where(m8, elv[j], lgv[j])
                plsc.store_scatter(
                    sm_v[b], [lax.broadcast(o + j, (LANES,)), col48],
                    mix * wvec)
            return gcarry

        lax.fori_loop(0, NG, group, 0)

    issue_in(0, 0)
    wait_in(0, 0)
    gat_copy(0).start()
    comp_fidx(0)
    sf_copy(0).start()

    def pair(i, carry):
        kk = 2 * i
        for b in range(2):
            k = kk + b
            q = 1 - b

            @pl.when(k + 1 < NCHUNK)
            def _():
                issue_in(k + 1, q)

            sf_copy(b).wait()
            insert_ellag(b)

            @pl.when(k + 1 < NCHUNK)
            def _():
                wait_in(k + 1, q)

                @pl.when(k >= 1)
                def _():
                    rows_copy(k - 1, q).wait()

                gat_copy(q).start()
                comp_fidx(q)

                @pl.when(k >= 1)
                def _():
                    sm_copy(k - 1, q).wait()

                sf_copy(q).start()

            gat_copy(b).wait()
            rows_copy(k, b).start()
            sm_copy(k, b).start()
        return carry

    lax.fori_loop(0, NPAIR, pair, 0)
    for k, b in ((NCHUNK - 2, 0), (NCHUNK - 1, 1)):
        rows_copy(k, b).wait()
        sm_copy(k, b).wait()


@jax.jit
def _run(ii, pp, ss, cc, tt, el, lg, item_t, part_t, sec_t, corr_t, time_t, w):
    mesh = plsc.VectorSubcoreMesh(core_axis_name="c", subcore_axis_name="s")
    dbl = lambda *a: [pltpu.VMEM(*a), pltpu.VMEM(*a)]
    f = pl.kernel(
        _body,
        out_type=jax.ShapeDtypeStruct((N, 128), jnp.float32),
        mesh=mesh,
        compiler_params=pltpu.CompilerParams(use_tc_tiling_on_sc=False,
                                            needs_layout_passes=False),
        scratch_types=[
            dbl((C,), jnp.int32),
            dbl((C,), jnp.int32),
            dbl((C,), jnp.int32),
            dbl((C,), jnp.int32),
            dbl((C,), jnp.int32),
            dbl((C,), jnp.float32),
            dbl((C,), jnp.float32),
            dbl((C,), jnp.int32),
            pltpu.VMEM((11, 17), jnp.float32),
            pltpu.VMEM((8, 17), jnp.float32),
            pltpu.VMEM((3, 9), jnp.float32),
            pltpu.VMEM((3, 9), jnp.float32),
            pltpu.VMEM((16,), jnp.float32),
            dbl((C, 64), jnp.float32),
            dbl((C, 64), jnp.float32),
            pltpu.VMEM_SHARED((NFPAD, 64), jnp.float32),
            [pltpu.SemaphoreType.DMA, pltpu.SemaphoreType.DMA],
            [pltpu.SemaphoreType.DMA, pltpu.SemaphoreType.DMA],
            [pltpu.SemaphoreType.DMA, pltpu.SemaphoreType.DMA],
            [pltpu.SemaphoreType.DMA, pltpu.SemaphoreType.DMA],
            [pltpu.SemaphoreType.DMA, pltpu.SemaphoreType.DMA],
        ],
    )
    return f(ii, pp, ss, cc, tt, el, lg, item_t, part_t, sec_t, corr_t, time_t, w)


def kernel(item_id, part_id, section, is_correct, timeliness,
           elapsed_time_norm, lag_time_norm,
           item_table, part_table, section_table,
           is_correct_table, timeliness_table, W_elapsed, W_lag):
    ii = item_id.reshape(N).astype(jnp.int32)
    pp = part_id.reshape(N).astype(jnp.int32)
    ss = section.reshape(N).astype(jnp.int32)
    cc = is_correct.reshape(N).astype(jnp.int32)
    tt = timeliness.reshape(N).astype(jnp.int32)
    el = elapsed_time_norm.reshape(N)
    lg = lag_time_norm.reshape(N)
    w = jnp.concatenate([W_elapsed.reshape(8), W_lag.reshape(8)])
    part_p = jnp.pad(part_table, ((0, 0), (0, 1)))
    sec_p = jnp.pad(section_table, ((0, 0), (0, 1)))
    corr_p = jnp.pad(is_correct_table, ((0, 0), (0, 1)))
    time_p = jnp.pad(timeliness_table, ((0, 0), (0, 1)))
    out = _run(ii, pp, ss, cc, tt, el, lg,
               item_table, part_p, sec_p, corr_p, time_p, w)
    return out.reshape(B, L, 128)


# speedup vs baseline: 1.1602x; 1.0025x over previous
import jax
import jax.numpy as jnp
from jax import lax
from jax.experimental import pallas as pl
from jax.experimental.pallas import tpu as pltpu
from jax.experimental.pallas import tpu_sc as plsc

B, L = 4096, 200
N = B * L
NC, NS, LANES = 2, 16, 16
NW = NC * NS
NTOK = N // NW
C = 400
NCHUNK = NTOK // C
NG = C // LANES
NPAIR = NCHUNK // 2
NF = 11 * 8 * 3 * 3
NFPAD = 1024


def _ge_count(x, step, n):
    acc = jnp.zeros_like(x)
    for k in range(1, n + 1):
        acc = acc + (x >= step * k).astype(jnp.int32)
    return acc


def _body(ii, pp, ss, cc, tt, el, lg,
          item_t, part_t, sec_t, corr_t, time_t, w,
          out,
          iidx_v, pp_v, ss_v, cc_v, tt_v, el_v, lg_v, fidx_v,
          part_v, sec_v, corr_v, time_v, w_v,
          rows_v, sm_v, f_sh,
          in_sem, gat_sem, sf_sem, out_sem, out2_sem):
    wid = lax.axis_index("s") * NC + lax.axis_index("c")
    base0 = wid * NTOK

    pltpu.sync_copy(part_t, part_v)
    pltpu.sync_copy(sec_t, sec_v)
    pltpu.sync_copy(corr_t, corr_v)
    pltpu.sync_copy(time_t, time_v)
    pltpu.sync_copy(w, w_v)
    wvec = w_v[...]

    sid0 = lax.axis_index("s")
    zofs = lax.iota(jnp.int32, LANES)
    for d in range(16):
        plsc.store_scatter(sm_v[0], [zofs, jnp.full((LANES,), 48 + d, jnp.int32)],
                           jnp.zeros((LANES,), jnp.float32))

    def fgroup(j, carry):
        g = sid0 + j * NS
        o = g * LANES
        f = jnp.minimum(lax.iota(jnp.int32, LANES) + o, NF - 1)
        p = _ge_count(f, 72, 10)
        r = f - p * 72
        s = _ge_count(r, 9, 7)
        r2 = r - s * 9
        c = _ge_count(r2, 3, 2)
        t = r2 - c * 3
        offs = lax.iota(jnp.int32, LANES)

        def flush(pairs):
            for v, col in pairs:
                plsc.store_scatter(
                    sm_v[0], [offs, jnp.full((LANES,), col, jnp.int32)], v)

        for d0 in range(0, 16, 8):
            pairs = []
            for d in range(d0, d0 + 8):
                dcol = jnp.full((LANES,), d, jnp.int32)
                pairs.append((plsc.load_gather(part_v, [p, dcol]), d))
                pairs.append((plsc.load_gather(sec_v, [s, dcol]), d + 16))
            flush(pairs)
        pairs = []
        for d in range(8):
            dcol = jnp.full((LANES,), d, jnp.int32)
            pairs.append((plsc.load_gather(corr_v, [c, dcol]), d + 32))
            pairs.append((plsc.load_gather(time_v, [t, dcol]), d + 40))
        flush(pairs)
        pltpu.sync_copy(sm_v[0].at[pl.ds(0, LANES)], f_sh.at[pl.ds(o, LANES)])
        return carry

    lax.fori_loop(0, NFPAD // (LANES * NS), fgroup, 0)

    plsc.subcore_barrier()

    def in_copies(k, b):
        base = base0 + k * C
        s = in_sem[b]
        return [
            pltpu.make_async_copy(ii.at[pl.ds(base, C)], iidx_v[b], s),
            pltpu.make_async_copy(pp.at[pl.ds(base, C)], pp_v[b], s),
            pltpu.make_async_copy(ss.at[pl.ds(base, C)], ss_v[b], s),
            pltpu.make_async_copy(cc.at[pl.ds(base, C)], cc_v[b], s),
            pltpu.make_async_copy(tt.at[pl.ds(base, C)], tt_v[b], s),
            pltpu.make_async_copy(el.at[pl.ds(base, C)], el_v[b], s),
            pltpu.make_async_copy(lg.at[pl.ds(base, C)], lg_v[b], s),
        ]

    def issue_in(k, b):
        for c in in_copies(k, b):
            c.start()

    def wait_in(k, b):
        for c in in_copies(k, b):
            c.wait()

    def gat_copy(b):
        return pltpu.make_async_copy(
            item_t.at[iidx_v[b]], rows_v[b], gat_sem[b])

    def sf_copy(b):
        return pltpu.make_async_copy(
            f_sh.at[fidx_v[b]], sm_v[b], sf_sem[b])

    def rows_copy(k, b):
        base = base0 + k * C
        return pltpu.make_async_copy(
            rows_v[b], out.at[pl.ds(base, C), pl.ds(0, 64)], out_sem[b])

    def sm_copy(k, b):
        base = base0 + k * C
        return pltpu.make_async_copy(
            sm_v[b], out.at[pl.ds(base, C), pl.ds(64, 64)], out2_sem[b])

    def comp_fidx(b):
        def group(g, gcarry):
            o = g * LANES
            pid = pp_v[b][pl.ds(o, LANES)]
            sid = ss_v[b][pl.ds(o, LANES)]
            cid = cc_v[b][pl.ds(o, LANES)]
            tid = tt_v[b][pl.ds(o, LANES)]
            fidx_v[b][pl.ds(o, LANES)] = pid * 72 + sid * 9 + cid * 3 + tid
            return gcarry

        lax.fori_loop(0, NG, group, 0)

    m8 = lax.iota(jnp.int32, LANES) < 8
    col48 = lax.iota(jnp.int32, LANES) + 48

    def insert_ellag(b):
        def group(g, gcarry):
            o = g * LANES
            elv = el_v[b][pl.ds(o, LANES)]
            lgv = lg_v[b][pl.ds(o, LANES)]
            for j in range(LANES):
                mix = jnp.where(m8, elv[j], lgv[j])
                plsc.store_scatter(
                    sm_v[b], [lax.broadcast(o + j, (LANES,)), col48],
                    mix * wvec)
            return gcarry

        lax.fori_loop(0, NG, group, 0)

    issue_in(0, 0)
    wait_in(0, 0)
    gat_copy(0).start()
    comp_fidx(0)
    sf_copy(0).start()

    def pair(i, carry):
        kk = 2 * i
        for b in range(2):
            k = kk + b
            q = 1 - b

            @pl.when(k + 1 < NCHUNK)
            def _():
                issue_in(k + 1, q)

            sf_copy(b).wait()
            insert_ellag(b)

            @pl.when(k + 1 < NCHUNK)
            def _():
                wait_in(k + 1, q)

                @pl.when(k >= 1)
                def _():
                    rows_copy(k - 1, q).wait()

                gat_copy(q).start()
                comp_fidx(q)

                @pl.when(k >= 1)
                def _():
                    sm_copy(k - 1, q).wait()

                sf_copy(q).start()

            gat_copy(b).wait()
            rows_copy(k, b).start()
            sm_copy(k, b).start()
        return carry

    lax.fori_loop(0, NPAIR, pair, 0)
    for k, b in ((NCHUNK - 2, 0), (NCHUNK - 1, 1)):
        rows_copy(k, b).wait()
        sm_copy(k, b).wait()


@jax.jit
def _run(ii, pp, ss, cc, tt, el, lg, item_t, part_t, sec_t, corr_t, time_t, w):
    mesh = plsc.VectorSubcoreMesh(core_axis_name="c", subcore_axis_name="s")
    dbl = lambda *a: [pltpu.VMEM(*a), pltpu.VMEM(*a)]
    f = pl.kernel(
        _body,
        out_type=jax.ShapeDtypeStruct((N, 128), jnp.float32),
        mesh=mesh,
        compiler_params=pltpu.CompilerParams(use_tc_tiling_on_sc=False,
                                            needs_layout_passes=False),
        scratch_types=[
            dbl((C,), jnp.int32),
            dbl((C,), jnp.int32),
            dbl((C,), jnp.int32),
            dbl((C,), jnp.int32),
            dbl((C,), jnp.int32),
            dbl((C,), jnp.float32),
            dbl((C,), jnp.float32),
            dbl((C,), jnp.int32),
            pltpu.VMEM((11, 17), jnp.float32),
            pltpu.VMEM((8, 17), jnp.float32),
            pltpu.VMEM((3, 9), jnp.float32),
            pltpu.VMEM((3, 9), jnp.float32),
            pltpu.VMEM((16,), jnp.float32),
            dbl((C, 64), jnp.float32),
            dbl((C, 64), jnp.float32),
            pltpu.VMEM_SHARED((NFPAD, 64), jnp.float32),
            [pltpu.SemaphoreType.DMA, pltpu.SemaphoreType.DMA],
            [pltpu.SemaphoreType.DMA, pltpu.SemaphoreType.DMA],
            [pltpu.SemaphoreType.DMA, pltpu.SemaphoreType.DMA],
            [pltpu.SemaphoreType.DMA, pltpu.SemaphoreType.DMA],
            [pltpu.SemaphoreType.DMA, pltpu.SemaphoreType.DMA],
        ],
    )
    return f(ii, pp, ss, cc, tt, el, lg, item_t, part_t, sec_t, corr_t, time_t, w)


def kernel(item_id, part_id, section, is_correct, timeliness,
           elapsed_time_norm, lag_time_norm,
           item_table, part_table, section_table,
           is_correct_table, timeliness_table, W_elapsed, W_lag):
    ii = item_id.reshape(N).astype(jnp.int32)
    pp = part_id.reshape(N).astype(jnp.int32)
    ss = section.reshape(N).astype(jnp.int32)
    cc = is_correct.reshape(N).astype(jnp.int32)
    tt = timeliness.reshape(N).astype(jnp.int32)
    el = elapsed_time_norm.reshape(N)
    lg = lag_time_norm.reshape(N)
    w = jnp.concatenate([W_elapsed.reshape(8), W_lag.reshape(8)])
    part_p = jnp.pad(part_table, ((0, 0), (0, 1)))
    sec_p = jnp.pad(section_table, ((0, 0), (0, 1)))
    corr_p = jnp.pad(is_correct_table, ((0, 0), (0, 1)))
    time_p = jnp.pad(timeliness_table, ((0, 0), (0, 1)))
    out = _run(ii, pp, ss, cc, tt, el, lg,
               item_table, part_p, sec_p, corr_p, time_p, w)
    return out.reshape(B, L, 128)
